# baseline (speedup 1.0000x reference)
import jax
import jax.numpy as jnp
from jax import lax
from jax.experimental import pallas as pl
from jax.experimental.pallas import tpu as pltpu
from jax.experimental.pallas import tpu_sc as plsc

_NUM_ITERS = 10
_ALPHA = 0.1
_N = 10000
_C = 128
_E = 320000

_NC = 2
_NS = 16
_NW = _NC * _NS
_EW = _E // _NW
_K = 80
_NCHUNK = _EW // _K
_NP = 10112
_RPT = _NP // _NS
_ZR = 8

_mesh = plsc.VectorSubcoreMesh(core_axis_name="c", subcore_axis_name="s")


def _zero_fill(zbuf, rows):
    def body(r, carry):
        for j in range(_C // 16):
            zbuf[r, pl.ds(j * 16, 16)] = jnp.zeros((16,), jnp.float32)
        return carry
    lax.fori_loop(0, rows, body, 0)


_NBUF = 3


def _agg_body(est_hbm, src_hbm, dst_hbm, out_hbm,
              srcs_v, dstb, rows, zbuf, acc, gsems, ssems, isems, zsem):
    cid = lax.axis_index("c")
    sid = lax.axis_index("s")
    wid = sid * _NC + cid
    row0 = sid * _RPT
    base0 = wid * _EW

    _zero_fill(zbuf, _ZR)
    def zfire(i, carry):
        pltpu.async_copy(zbuf, acc.at[pl.ds(row0 + i * _ZR, _ZR)], zsem)
        return carry
    lax.fori_loop(0, _RPT // _ZR, zfire, 0)
    pltpu.sync_copy(src_hbm.at[pl.ds(base0, _EW)], srcs_v)
    def zdrain(i, carry):
        pltpu.make_async_copy(zbuf, acc.at[pl.ds(row0 + i * _ZR, _ZR)], zsem).wait()
        return carry
    lax.fori_loop(0, _RPT // _ZR, zdrain, 0)
    plsc.subcore_barrier()

    def ifire(c, b):
        pltpu.async_copy(dst_hbm.at[pl.ds(base0 + c * _K, _K)], dstb[b], isems[b])
    def iwait(c, b):
        pltpu.make_async_copy(dst_hbm.at[pl.ds(base0 + c * _K, _K)], dstb[b],
                              isems[b]).wait()
    def gfire(c, b):
        pltpu.async_copy(est_hbm.at[srcs_v.at[pl.ds(c * _K, _K)]], rows[b], gsems[b],
                         priority=1)
    def gwait(c, b):
        pltpu.make_async_copy(est_hbm.at[srcs_v.at[pl.ds(c * _K, _K)]], rows[b],
                              gsems[b]).wait()
    def sfire(b):
        pltpu.async_copy(rows[b], acc.at[dstb[b]], ssems[b], add=True)
    def swait(b):
        pltpu.make_async_copy(rows[b], acc.at[dstb[b]], ssems[b]).wait()

    for b in range(_NBUF):
        ifire(b, b)
        gfire(b, b)

    last = _NCHUNK - 1
    nbody = _NCHUNK // _NBUF
    ntail = _NCHUNK - nbody * _NBUF
    def body(g, carry):
        c0 = _NBUF * g
        for b in range(_NBUF):
            iwait(c0 + b, b)
            gwait(c0 + b, b)
            sfire(b)
        for b in range(_NBUF):
            swait(b)
            cn = jnp.minimum(c0 + _NBUF + b, last)
            ifire(cn, b)
            gfire(cn, b)
        return carry
    lax.fori_loop(0, nbody, body, 0)

    for t in range(ntail):
        iwait(nbody * _NBUF + t, t)
        gwait(nbody * _NBUF + t, t)
        sfire(t)
    for b in range(ntail, _NBUF):
        iwait(last, b)
        gwait(last, b)
    for t in range(ntail):
        swait(t)
    plsc.subcore_barrier()

    pltpu.sync_copy(acc.at[pl.ds(row0, _RPT)], out_hbm.at[cid, pl.ds(row0, _RPT)])


_agg = pl.kernel(
    _agg_body,
    out_type=jax.ShapeDtypeStruct((_NC, _NP, _C), jnp.float32),
    mesh=_mesh,
    scratch_types=[
        pltpu.VMEM((_EW,), jnp.int32),
        [pltpu.VMEM((_K,), jnp.int32)] * _NBUF,
        [pltpu.VMEM((_K, _C), jnp.float32)] * _NBUF,
        pltpu.VMEM((_ZR, _C), jnp.float32),
        pltpu.VMEM_SHARED((_NP, _C), jnp.float32),
        [pltpu.SemaphoreType.DMA] * _NBUF,
        [pltpu.SemaphoreType.DMA] * _NBUF,
        [pltpu.SemaphoreType.DMA] * _NBUF,
        pltpu.SemaphoreType.DMA,
    ],
)


_WD = _C


def _deg_body(dst3_hbm, out_hbm, dsts_v, ones_v, zbuf, acc, ssem, zsem):
    cid = lax.axis_index("c")
    sid = lax.axis_index("s")
    wid = sid * _NC + cid
    row0 = sid * _RPT

    _zero_fill(zbuf, _ZR)
    def ofill(r, carry):
        for j in range(_WD // 16):
            ones_v[r, pl.ds(j * 16, 16)] = jnp.ones((16,), jnp.float32)
        return carry
    lax.fori_loop(0, _K, ofill, 0)

    def zfire(i, carry):
        pltpu.async_copy(zbuf, acc.at[pl.ds(row0 + i * _ZR, _ZR)], zsem)
        return carry
    lax.fori_loop(0, _RPT // _ZR, zfire, 0)
    pltpu.sync_copy(dst3_hbm.at[wid], dsts_v)
    def zdrain(i, carry):
        pltpu.make_async_copy(zbuf, acc.at[pl.ds(row0 + i * _ZR, _ZR)], zsem).wait()
        return carry
    lax.fori_loop(0, _RPT // _ZR, zdrain, 0)
    plsc.subcore_barrier()

    def sfire(c, carry):
        pltpu.async_copy(ones_v, acc.at[dsts_v.at[c]], ssem, add=True)
        return carry
    lax.fori_loop(0, _NCHUNK, sfire, 0)
    def sdrain(c, carry):
        pltpu.make_async_copy(ones_v, acc.at[dsts_v.at[c]], ssem).wait()
        return carry
    lax.fori_loop(0, _NCHUNK, sdrain, 0)
    plsc.subcore_barrier()

    pltpu.sync_copy(acc.at[pl.ds(row0, _RPT)], out_hbm.at[cid, pl.ds(row0, _RPT)])


_deg = pl.kernel(
    _deg_body,
    out_type=jax.ShapeDtypeStruct((_NC, _NP, _WD), jnp.float32),
    mesh=_mesh,
    scratch_types=[
        pltpu.VMEM((_NCHUNK, _K), jnp.int32),
        pltpu.VMEM((_K, _WD), jnp.float32),
        pltpu.VMEM((_ZR, _WD), jnp.float32),
        pltpu.VMEM_SHARED((_NP, _WD), jnp.float32),
        pltpu.SemaphoreType.DMA,
        pltpu.SemaphoreType.DMA,
    ],
)


def _softmax_body(w_ref, p_ref):
    w = w_ref[...]
    m = jnp.max(w, axis=1, keepdims=True)
    e = jnp.exp(w - m)
    p_ref[...] = e / jnp.sum(e, axis=1, keepdims=True)


_softmax = pl.pallas_call(
    _softmax_body,
    out_shape=jax.ShapeDtypeStruct((_C, _C), jnp.float32),
)

_RB = 1264


def _norm_body(degp_ref, norm_ref):
    d = degp_ref[0] + degp_ref[1]
    n1 = 1.0 / jnp.maximum(d[:, 0:1], 1.0)
    norm_ref[...] = jnp.broadcast_to(n1, (_RB, _C))


_norm = pl.pallas_call(
    _norm_body,
    grid=(_NP // _RB,),
    in_specs=[pl.BlockSpec((_NC, _RB, _WD), lambda i: (0, i, 0))],
    out_specs=pl.BlockSpec((_RB, _C), lambda i: (i, 0)),
    out_shape=jax.ShapeDtypeStruct((_NP, _C), jnp.float32),
)


def _step_body(sp_ref, normf_ref, est0_ref, p_ref, out_ref):
    s = sp_ref[0] + sp_ref[1]
    m = jnp.dot(s, p_ref[...], preferred_element_type=jnp.float32)
    out_ref[...] = (1.0 - _ALPHA) * normf_ref[...] * m + _ALPHA * est0_ref[...]


_step = pl.pallas_call(
    _step_body,
    grid=(_NP // _RB,),
    in_specs=[
        pl.BlockSpec((_NC, _RB, _C), lambda i: (0, i, 0)),
        pl.BlockSpec((_RB, _C), lambda i: (i, 0)),
        pl.BlockSpec((_RB, _C), lambda i: (i, 0)),
        pl.BlockSpec((_C, _C), lambda i: (0, 0)),
    ],
    out_specs=pl.BlockSpec((_RB, _C), lambda i: (i, 0)),
    out_shape=jax.ShapeDtypeStruct((_NP, _C), jnp.float32),
)


def kernel(edge_index, estimates, W):
    src = edge_index[0]
    dst = edge_index[1]
    dst3 = dst.reshape(_NW, _NCHUNK, _K)
    P = _softmax(W)
    degp = _deg(dst3)
    normf = _norm(degp)
    est0 = jnp.pad(estimates, ((0, _NP - _N), (0, 0)))
    est = est0
    for _ in range(_NUM_ITERS):
        sp = _agg(est, src, dst)
        est = _step(sp, normf, est0, P)
    return est[:_N]


# speedup vs baseline: 1.0503x; 1.0503x over previous
import jax
import jax.numpy as jnp
from jax import lax
from jax.experimental import pallas as pl
from jax.experimental.pallas import tpu as pltpu
from jax.experimental.pallas import tpu_sc as plsc

_NUM_ITERS = 10
_ALPHA = 0.1
_N = 10000
_C = 128
_E = 320000

_NC = 2
_NS = 16
_NW = _NC * _NS
_EW = _E // _NW
_K = 40
_NCHUNK = _EW // _K
_NP = 10112
_RPT = _NP // _NS
_ZR = 8

_mesh = plsc.VectorSubcoreMesh(core_axis_name="c", subcore_axis_name="s")


def _zero_fill(zbuf, rows):
    def body(r, carry):
        for j in range(_C // 16):
            zbuf[r, pl.ds(j * 16, 16)] = jnp.zeros((16,), jnp.float32)
        return carry
    lax.fori_loop(0, rows, body, 0)


_NBUF = 5


def _agg_body(est_hbm, src_hbm, dst_hbm, out_hbm,
              srcs_v, dstb, rows, zbuf, acc, gsems, ssems, isems, zsem):
    cid = lax.axis_index("c")
    sid = lax.axis_index("s")
    wid = sid * _NC + cid
    row0 = sid * _RPT
    base0 = wid * _EW

    _zero_fill(zbuf, _ZR)
    def zfire(i, carry):
        pltpu.async_copy(zbuf, acc.at[pl.ds(row0 + i * _ZR, _ZR)], zsem)
        return carry
    lax.fori_loop(0, _RPT // _ZR, zfire, 0)
    pltpu.sync_copy(src_hbm.at[pl.ds(base0, _EW)], srcs_v)
    def zdrain(i, carry):
        pltpu.make_async_copy(zbuf, acc.at[pl.ds(row0 + i * _ZR, _ZR)], zsem).wait()
        return carry
    lax.fori_loop(0, _RPT // _ZR, zdrain, 0)
    plsc.subcore_barrier()

    def ifire(c, b):
        pltpu.async_copy(dst_hbm.at[pl.ds(base0 + c * _K, _K)], dstb[b], isems[b])
    def iwait(c, b):
        pltpu.make_async_copy(dst_hbm.at[pl.ds(base0 + c * _K, _K)], dstb[b],
                              isems[b]).wait()
    def gfire(c, b):
        pltpu.async_copy(est_hbm.at[srcs_v.at[pl.ds(c * _K, _K)]], rows[b], gsems[b])
    def gwait(c, b):
        pltpu.make_async_copy(est_hbm.at[srcs_v.at[pl.ds(c * _K, _K)]], rows[b],
                              gsems[b]).wait()
    def sfire(b):
        pltpu.async_copy(rows[b], acc.at[dstb[b]], ssems[b], add=True)
    def swait(b):
        pltpu.make_async_copy(rows[b], acc.at[dstb[b]], ssems[b]).wait()

    for b in range(_NBUF):
        ifire(b, b)
        gfire(b, b)

    last = _NCHUNK - 1
    nbody = _NCHUNK // _NBUF
    ntail = _NCHUNK - nbody * _NBUF
    def body(g, carry):
        c0 = _NBUF * g
        for b in range(_NBUF):
            iwait(c0 + b, b)
            gwait(c0 + b, b)
            sfire(b)
        for b in range(_NBUF):
            swait(b)
            cn = jnp.minimum(c0 + _NBUF + b, last)
            ifire(cn, b)
            gfire(cn, b)
        return carry
    lax.fori_loop(0, nbody, body, 0)

    for t in range(ntail):
        iwait(nbody * _NBUF + t, t)
        gwait(nbody * _NBUF + t, t)
        sfire(t)
    for b in range(ntail, _NBUF):
        iwait(last, b)
        gwait(last, b)
    for t in range(ntail):
        swait(t)
    plsc.subcore_barrier()

    pltpu.sync_copy(acc.at[pl.ds(row0, _RPT)], out_hbm.at[cid, pl.ds(row0, _RPT)])


_agg = pl.kernel(
    _agg_body,
    out_type=jax.ShapeDtypeStruct((_NC, _NP, _C), jnp.float32),
    mesh=_mesh,
    scratch_types=[
        pltpu.VMEM((_EW,), jnp.int32),
        [pltpu.VMEM((_K,), jnp.int32)] * _NBUF,
        [pltpu.VMEM((_K, _C), jnp.float32)] * _NBUF,
        pltpu.VMEM((_ZR, _C), jnp.float32),
        pltpu.VMEM_SHARED((_NP, _C), jnp.float32),
        [pltpu.SemaphoreType.DMA] * _NBUF,
        [pltpu.SemaphoreType.DMA] * _NBUF,
        [pltpu.SemaphoreType.DMA] * _NBUF,
        pltpu.SemaphoreType.DMA,
    ],
)


_WD = _C


def _deg_body(dst3_hbm, out_hbm, dsts_v, ones_v, zbuf, acc, ssem, zsem):
    cid = lax.axis_index("c")
    sid = lax.axis_index("s")
    wid = sid * _NC + cid
    row0 = sid * _RPT

    _zero_fill(zbuf, _ZR)
    def ofill(r, carry):
        for j in range(_WD // 16):
            ones_v[r, pl.ds(j * 16, 16)] = jnp.ones((16,), jnp.float32)
        return carry
    lax.fori_loop(0, _K, ofill, 0)

    def zfire(i, carry):
        pltpu.async_copy(zbuf, acc.at[pl.ds(row0 + i * _ZR, _ZR)], zsem)
        return carry
    lax.fori_loop(0, _RPT // _ZR, zfire, 0)
    pltpu.sync_copy(dst3_hbm.at[wid], dsts_v)
    def zdrain(i, carry):
        pltpu.make_async_copy(zbuf, acc.at[pl.ds(row0 + i * _ZR, _ZR)], zsem).wait()
        return carry
    lax.fori_loop(0, _RPT // _ZR, zdrain, 0)
    plsc.subcore_barrier()

    def sfire(c, carry):
        pltpu.async_copy(ones_v, acc.at[dsts_v.at[c]], ssem, add=True)
        return carry
    lax.fori_loop(0, _NCHUNK, sfire, 0)
    def sdrain(c, carry):
        pltpu.make_async_copy(ones_v, acc.at[dsts_v.at[c]], ssem).wait()
        return carry
    lax.fori_loop(0, _NCHUNK, sdrain, 0)
    plsc.subcore_barrier()

    pltpu.sync_copy(acc.at[pl.ds(row0, _RPT)], out_hbm.at[cid, pl.ds(row0, _RPT)])


_deg = pl.kernel(
    _deg_body,
    out_type=jax.ShapeDtypeStruct((_NC, _NP, _WD), jnp.float32),
    mesh=_mesh,
    scratch_types=[
        pltpu.VMEM((_NCHUNK, _K), jnp.int32),
        pltpu.VMEM((_K, _WD), jnp.float32),
        pltpu.VMEM((_ZR, _WD), jnp.float32),
        pltpu.VMEM_SHARED((_NP, _WD), jnp.float32),
        pltpu.SemaphoreType.DMA,
        pltpu.SemaphoreType.DMA,
    ],
)


def _softmax_body(w_ref, p_ref):
    w = w_ref[...]
    m = jnp.max(w, axis=1, keepdims=True)
    e = jnp.exp(w - m)
    p_ref[...] = e / jnp.sum(e, axis=1, keepdims=True)


_softmax = pl.pallas_call(
    _softmax_body,
    out_shape=jax.ShapeDtypeStruct((_C, _C), jnp.float32),
)

_RB = 1264


def _norm_body(degp_ref, norm_ref):
    d = degp_ref[0] + degp_ref[1]
    n1 = 1.0 / jnp.maximum(d[:, 0:1], 1.0)
    norm_ref[...] = jnp.broadcast_to(n1, (_RB, _C))


_norm = pl.pallas_call(
    _norm_body,
    grid=(_NP // _RB,),
    in_specs=[pl.BlockSpec((_NC, _RB, _WD), lambda i: (0, i, 0))],
    out_specs=pl.BlockSpec((_RB, _C), lambda i: (i, 0)),
    out_shape=jax.ShapeDtypeStruct((_NP, _C), jnp.float32),
)


def _step_body(sp_ref, normf_ref, est0_ref, p_ref, out_ref):
    s = sp_ref[0] + sp_ref[1]
    m = jnp.dot(s, p_ref[...], preferred_element_type=jnp.float32)
    out_ref[...] = (1.0 - _ALPHA) * normf_ref[...] * m + _ALPHA * est0_ref[...]


_step = pl.pallas_call(
    _step_body,
    grid=(_NP // _RB,),
    in_specs=[
        pl.BlockSpec((_NC, _RB, _C), lambda i: (0, i, 0)),
        pl.BlockSpec((_RB, _C), lambda i: (i, 0)),
        pl.BlockSpec((_RB, _C), lambda i: (i, 0)),
        pl.BlockSpec((_C, _C), lambda i: (0, 0)),
    ],
    out_specs=pl.BlockSpec((_RB, _C), lambda i: (i, 0)),
    out_shape=jax.ShapeDtypeStruct((_NP, _C), jnp.float32),
)


def kernel(edge_index, estimates, W):
    src = edge_index[0]
    dst = edge_index[1]
    dst3 = dst.reshape(_NW, _NCHUNK, _K)
    P = _softmax(W)
    degp = _deg(dst3)
    normf = _norm(degp)
    est0 = jnp.pad(estimates, ((0, _NP - _N), (0, 0)))
    est = est0
    for _ in range(_NUM_ITERS):
        sp = _agg(est, src, dst)
        est = _step(sp, normf, est0, P)
    return est[:_N]


# speedup vs baseline: 1.0827x; 1.0309x over previous
import jax
import jax.numpy as jnp
from jax import lax
from jax.experimental import pallas as pl
from jax.experimental.pallas import tpu as pltpu
from jax.experimental.pallas import tpu_sc as plsc

_NUM_ITERS = 10
_ALPHA = 0.1
_N = 10000
_C = 128
_E = 320000

_NC = 2
_NS = 16
_NW = _NC * _NS
_EW = _E // _NW
_K = 40
_NCHUNK = _EW // _K
_NP = 10112
_RPT = _NP // _NS
_ZR = 8

_mesh = plsc.VectorSubcoreMesh(core_axis_name="c", subcore_axis_name="s")


def _zero_fill(zbuf, rows):
    def body(r, carry):
        for j in range(_C // 16):
            zbuf[r, pl.ds(j * 16, 16)] = jnp.zeros((16,), jnp.float32)
        return carry
    lax.fori_loop(0, rows, body, 0)


_NBUF = 6


def _agg_body(est_hbm, src_hbm, dst_hbm, out_hbm,
              srcs_v, dstb, rows, zbuf, acc, gsems, ssems, isems, zsem):
    cid = lax.axis_index("c")
    sid = lax.axis_index("s")
    wid = sid * _NC + cid
    row0 = sid * _RPT
    base0 = wid * _EW

    _zero_fill(zbuf, _ZR)
    def zfire(i, carry):
        pltpu.async_copy(zbuf, acc.at[pl.ds(row0 + i * _ZR, _ZR)], zsem)
        return carry
    lax.fori_loop(0, _RPT // _ZR, zfire, 0)
    pltpu.sync_copy(src_hbm.at[pl.ds(base0, _EW)], srcs_v)
    def zdrain(i, carry):
        pltpu.make_async_copy(zbuf, acc.at[pl.ds(row0 + i * _ZR, _ZR)], zsem).wait()
        return carry
    lax.fori_loop(0, _RPT // _ZR, zdrain, 0)
    plsc.subcore_barrier()

    def ifire(c, b):
        pltpu.async_copy(dst_hbm.at[pl.ds(base0 + c * _K, _K)], dstb[b], isems[b])
    def iwait(c, b):
        pltpu.make_async_copy(dst_hbm.at[pl.ds(base0 + c * _K, _K)], dstb[b],
                              isems[b]).wait()
    def gfire(c, b):
        pltpu.async_copy(est_hbm.at[srcs_v.at[pl.ds(c * _K, _K)]], rows[b], gsems[b])
    def gwait(c, b):
        pltpu.make_async_copy(est_hbm.at[srcs_v.at[pl.ds(c * _K, _K)]], rows[b],
                              gsems[b]).wait()
    def sfire(b):
        pltpu.async_copy(rows[b], acc.at[dstb[b]], ssems[b], add=True)
    def swait(b):
        pltpu.make_async_copy(rows[b], acc.at[dstb[b]], ssems[b]).wait()

    for b in range(_NBUF):
        ifire(b, b)
        gfire(b, b)

    last = _NCHUNK - 1
    nbody = _NCHUNK // _NBUF
    ntail = _NCHUNK - nbody * _NBUF
    def body(g, carry):
        c0 = _NBUF * g
        for b in range(_NBUF):
            iwait(c0 + b, b)
            gwait(c0 + b, b)
            sfire(b)
        for b in range(_NBUF):
            swait(b)
            cn = jnp.minimum(c0 + _NBUF + b, last)
            ifire(cn, b)
            gfire(cn, b)
        return carry
    lax.fori_loop(0, nbody, body, 0)

    for t in range(ntail):
        iwait(nbody * _NBUF + t, t)
        gwait(nbody * _NBUF + t, t)
        sfire(t)
    for b in range(ntail, _NBUF):
        iwait(last, b)
        gwait(last, b)
    for t in range(ntail):
        swait(t)
    plsc.subcore_barrier()

    pltpu.sync_copy(acc.at[pl.ds(row0, _RPT)], out_hbm.at[cid, pl.ds(row0, _RPT)])


_agg = pl.kernel(
    _agg_body,
    out_type=jax.ShapeDtypeStruct((_NC, _NP, _C), jnp.float32),
    mesh=_mesh,
    scratch_types=[
        pltpu.VMEM((_EW,), jnp.int32),
        [pltpu.VMEM((_K,), jnp.int32)] * _NBUF,
        [pltpu.VMEM((_K, _C), jnp.float32)] * _NBUF,
        pltpu.VMEM((_ZR, _C), jnp.float32),
        pltpu.VMEM_SHARED((_NP, _C), jnp.float32),
        [pltpu.SemaphoreType.DMA] * _NBUF,
        [pltpu.SemaphoreType.DMA] * _NBUF,
        [pltpu.SemaphoreType.DMA] * _NBUF,
        pltpu.SemaphoreType.DMA,
    ],
)


_WD = _C


def _deg_body(dst3_hbm, out_hbm, dsts_v, ones_v, zbuf, acc, ssem, zsem):
    cid = lax.axis_index("c")
    sid = lax.axis_index("s")
    wid = sid * _NC + cid
    row0 = sid * _RPT

    _zero_fill(zbuf, _ZR)
    def ofill(r, carry):
        for j in range(_WD // 16):
            ones_v[r, pl.ds(j * 16, 16)] = jnp.ones((16,), jnp.float32)
        return carry
    lax.fori_loop(0, _K, ofill, 0)

    def zfire(i, carry):
        pltpu.async_copy(zbuf, acc.at[pl.ds(row0 + i * _ZR, _ZR)], zsem)
        return carry
    lax.fori_loop(0, _RPT // _ZR, zfire, 0)
    pltpu.sync_copy(dst3_hbm.at[wid], dsts_v)
    def zdrain(i, carry):
        pltpu.make_async_copy(zbuf, acc.at[pl.ds(row0 + i * _ZR, _ZR)], zsem).wait()
        return carry
    lax.fori_loop(0, _RPT // _ZR, zdrain, 0)
    plsc.subcore_barrier()

    def sfire(c, carry):
        pltpu.async_copy(ones_v, acc.at[dsts_v.at[c]], ssem, add=True)
        return carry
    lax.fori_loop(0, _NCHUNK, sfire, 0)
    def sdrain(c, carry):
        pltpu.make_async_copy(ones_v, acc.at[dsts_v.at[c]], ssem).wait()
        return carry
    lax.fori_loop(0, _NCHUNK, sdrain, 0)
    plsc.subcore_barrier()

    pltpu.sync_copy(acc.at[pl.ds(row0, _RPT)], out_hbm.at[cid, pl.ds(row0, _RPT)])


_deg = pl.kernel(
    _deg_body,
    out_type=jax.ShapeDtypeStruct((_NC, _NP, _WD), jnp.float32),
    mesh=_mesh,
    scratch_types=[
        pltpu.VMEM((_NCHUNK, _K), jnp.int32),
        pltpu.VMEM((_K, _WD), jnp.float32),
        pltpu.VMEM((_ZR, _WD), jnp.float32),
        pltpu.VMEM_SHARED((_NP, _WD), jnp.float32),
        pltpu.SemaphoreType.DMA,
        pltpu.SemaphoreType.DMA,
    ],
)


def _softmax_body(w_ref, p_ref):
    w = w_ref[...]
    m = jnp.max(w, axis=1, keepdims=True)
    e = jnp.exp(w - m)
    p_ref[...] = e / jnp.sum(e, axis=1, keepdims=True)


_softmax = pl.pallas_call(
    _softmax_body,
    out_shape=jax.ShapeDtypeStruct((_C, _C), jnp.float32),
)

_RB = 1264


def _norm_body(degp_ref, norm_ref):
    d = degp_ref[0] + degp_ref[1]
    n1 = 1.0 / jnp.maximum(d[:, 0:1], 1.0)
    norm_ref[...] = jnp.broadcast_to(n1, (_RB, _C))


_norm = pl.pallas_call(
    _norm_body,
    grid=(_NP // _RB,),
    in_specs=[pl.BlockSpec((_NC, _RB, _WD), lambda i: (0, i, 0))],
    out_specs=pl.BlockSpec((_RB, _C), lambda i: (i, 0)),
    out_shape=jax.ShapeDtypeStruct((_NP, _C), jnp.float32),
)


def _step_body(sp_ref, normf_ref, est0_ref, p_ref, out_ref):
    s = sp_ref[0] + sp_ref[1]
    m = jnp.dot(s, p_ref[...], preferred_element_type=jnp.float32)
    out_ref[...] = (1.0 - _ALPHA) * normf_ref[...] * m + _ALPHA * est0_ref[...]


_step = pl.pallas_call(
    _step_body,
    grid=(_NP // _RB,),
    in_specs=[
        pl.BlockSpec((_NC, _RB, _C), lambda i: (0, i, 0)),
        pl.BlockSpec((_RB, _C), lambda i: (i, 0)),
        pl.BlockSpec((_RB, _C), lambda i: (i, 0)),
        pl.BlockSpec((_C, _C), lambda i: (0, 0)),
    ],
    out_specs=pl.BlockSpec((_RB, _C), lambda i: (i, 0)),
    out_shape=jax.ShapeDtypeStruct((_NP, _C), jnp.float32),
)


def kernel(edge_index, estimates, W):
    src = edge_index[0]
    dst = edge_index[1]
    dst3 = dst.reshape(_NW, _NCHUNK, _K)
    P = _softmax(W)
    degp = _deg(dst3)
    normf = _norm(degp)
    est0 = jnp.pad(estimates, ((0, _NP - _N), (0, 0)))
    est = est0
    for _ in range(_NUM_ITERS):
        sp = _agg(est, src, dst)
        est = _step(sp, normf, est0, P)
    return est[:_N]


# speedup vs baseline: 1.0912x; 1.0078x over previous
import jax
import jax.numpy as jnp
from jax import lax
from jax.experimental import pallas as pl
from jax.experimental.pallas import tpu as pltpu
from jax.experimental.pallas import tpu_sc as plsc

_NUM_ITERS = 10
_ALPHA = 0.1
_N = 10000
_C = 128
_E = 320000

_NC = 2
_NS = 16
_NW = _NC * _NS
_EW = _E // _NW
_K = 40
_NCHUNK = _EW // _K
_NP = 10112
_RPT = _NP // _NS
_ZR = 8

_mesh = plsc.VectorSubcoreMesh(core_axis_name="c", subcore_axis_name="s")


def _zero_fill(zbuf, rows):
    def body(r, carry):
        for j in range(_C // 16):
            zbuf[r, pl.ds(j * 16, 16)] = jnp.zeros((16,), jnp.float32)
        return carry
    lax.fori_loop(0, rows, body, 0)


_NBUF = 7


def _agg_body(est_hbm, src_hbm, dst_hbm, out_hbm,
              srcs_v, dstb, rows, zbuf, acc, gsems, ssems, isems, zsem):
    cid = lax.axis_index("c")
    sid = lax.axis_index("s")
    wid = sid * _NC + cid
    row0 = sid * _RPT
    base0 = wid * _EW

    _zero_fill(zbuf, _ZR)
    def zfire(i, carry):
        pltpu.async_copy(zbuf, acc.at[pl.ds(row0 + i * _ZR, _ZR)], zsem)
        return carry
    lax.fori_loop(0, _RPT // _ZR, zfire, 0)
    pltpu.sync_copy(src_hbm.at[pl.ds(base0, _EW)], srcs_v)
    def zdrain(i, carry):
        pltpu.make_async_copy(zbuf, acc.at[pl.ds(row0 + i * _ZR, _ZR)], zsem).wait()
        return carry
    lax.fori_loop(0, _RPT // _ZR, zdrain, 0)
    plsc.subcore_barrier()

    def ifire(c, b):
        pltpu.async_copy(dst_hbm.at[pl.ds(base0 + c * _K, _K)], dstb[b], isems[b])
    def iwait(c, b):
        pltpu.make_async_copy(dst_hbm.at[pl.ds(base0 + c * _K, _K)], dstb[b],
                              isems[b]).wait()
    def gfire(c, b):
        pltpu.async_copy(est_hbm.at[srcs_v.at[pl.ds(c * _K, _K)]], rows[b], gsems[b])
    def gwait(c, b):
        pltpu.make_async_copy(est_hbm.at[srcs_v.at[pl.ds(c * _K, _K)]], rows[b],
                              gsems[b]).wait()
    def sfire(b):
        pltpu.async_copy(rows[b], acc.at[dstb[b]], ssems[b], add=True)
    def swait(b):
        pltpu.make_async_copy(rows[b], acc.at[dstb[b]], ssems[b]).wait()

    for b in range(_NBUF):
        ifire(b, b)
        gfire(b, b)

    last = _NCHUNK - 1
    nbody = _NCHUNK // _NBUF
    ntail = _NCHUNK - nbody * _NBUF
    def body(g, carry):
        c0 = _NBUF * g
        for b in range(_NBUF):
            iwait(c0 + b, b)
            gwait(c0 + b, b)
            sfire(b)
        for b in range(_NBUF):
            swait(b)
            cn = jnp.minimum(c0 + _NBUF + b, last)
            ifire(cn, b)
            gfire(cn, b)
        return carry
    lax.fori_loop(0, nbody, body, 0)

    for t in range(ntail):
        iwait(nbody * _NBUF + t, t)
        gwait(nbody * _NBUF + t, t)
        sfire(t)
    for b in range(ntail, _NBUF):
        iwait(last, b)
        gwait(last, b)
    for t in range(ntail):
        swait(t)
    plsc.subcore_barrier()

    pltpu.sync_copy(acc.at[pl.ds(row0, _RPT)], out_hbm.at[cid, pl.ds(row0, _RPT)])


_agg = pl.kernel(
    _agg_body,
    out_type=jax.ShapeDtypeStruct((_NC, _NP, _C), jnp.float32),
    mesh=_mesh,
    scratch_types=[
        pltpu.VMEM((_EW,), jnp.int32),
        [pltpu.VMEM((_K,), jnp.int32)] * _NBUF,
        [pltpu.VMEM((_K, _C), jnp.float32)] * _NBUF,
        pltpu.VMEM((_ZR, _C), jnp.float32),
        pltpu.VMEM_SHARED((_NP, _C), jnp.float32),
        [pltpu.SemaphoreType.DMA] * _NBUF,
        [pltpu.SemaphoreType.DMA] * _NBUF,
        [pltpu.SemaphoreType.DMA] * _NBUF,
        pltpu.SemaphoreType.DMA,
    ],
)


_WD = _C


def _deg_body(dst3_hbm, out_hbm, dsts_v, ones_v, zbuf, acc, ssem, zsem):
    cid = lax.axis_index("c")
    sid = lax.axis_index("s")
    wid = sid * _NC + cid
    row0 = sid * _RPT

    _zero_fill(zbuf, _ZR)
    def ofill(r, carry):
        for j in range(_WD // 16):
            ones_v[r, pl.ds(j * 16, 16)] = jnp.ones((16,), jnp.float32)
        return carry
    lax.fori_loop(0, _K, ofill, 0)

    def zfire(i, carry):
        pltpu.async_copy(zbuf, acc.at[pl.ds(row0 + i * _ZR, _ZR)], zsem)
        return carry
    lax.fori_loop(0, _RPT // _ZR, zfire, 0)
    pltpu.sync_copy(dst3_hbm.at[wid], dsts_v)
    def zdrain(i, carry):
        pltpu.make_async_copy(zbuf, acc.at[pl.ds(row0 + i * _ZR, _ZR)], zsem).wait()
        return carry
    lax.fori_loop(0, _RPT // _ZR, zdrain, 0)
    plsc.subcore_barrier()

    def sfire(c, carry):
        pltpu.async_copy(ones_v, acc.at[dsts_v.at[c]], ssem, add=True)
        return carry
    lax.fori_loop(0, _NCHUNK, sfire, 0)
    def sdrain(c, carry):
        pltpu.make_async_copy(ones_v, acc.at[dsts_v.at[c]], ssem).wait()
        return carry
    lax.fori_loop(0, _NCHUNK, sdrain, 0)
    plsc.subcore_barrier()

    pltpu.sync_copy(acc.at[pl.ds(row0, _RPT)], out_hbm.at[cid, pl.ds(row0, _RPT)])


_deg = pl.kernel(
    _deg_body,
    out_type=jax.ShapeDtypeStruct((_NC, _NP, _WD), jnp.float32),
    mesh=_mesh,
    scratch_types=[
        pltpu.VMEM((_NCHUNK, _K), jnp.int32),
        pltpu.VMEM((_K, _WD), jnp.float32),
        pltpu.VMEM((_ZR, _WD), jnp.float32),
        pltpu.VMEM_SHARED((_NP, _WD), jnp.float32),
        pltpu.SemaphoreType.DMA,
        pltpu.SemaphoreType.DMA,
    ],
)


def _softmax_body(w_ref, p_ref):
    w = w_ref[...]
    m = jnp.max(w, axis=1, keepdims=True)
    e = jnp.exp(w - m)
    p_ref[...] = e / jnp.sum(e, axis=1, keepdims=True)


_softmax = pl.pallas_call(
    _softmax_body,
    out_shape=jax.ShapeDtypeStruct((_C, _C), jnp.float32),
)

_RB = 1264


def _norm_body(degp_ref, norm_ref):
    d = degp_ref[0] + degp_ref[1]
    n1 = 1.0 / jnp.maximum(d[:, 0:1], 1.0)
    norm_ref[...] = jnp.broadcast_to(n1, (_RB, _C))


_norm = pl.pallas_call(
    _norm_body,
    grid=(_NP // _RB,),
    in_specs=[pl.BlockSpec((_NC, _RB, _WD), lambda i: (0, i, 0))],
    out_specs=pl.BlockSpec((_RB, _C), lambda i: (i, 0)),
    out_shape=jax.ShapeDtypeStruct((_NP, _C), jnp.float32),
)


def _step_body(sp_ref, normf_ref, est0_ref, p_ref, out_ref):
    s = sp_ref[0] + sp_ref[1]
    m = jnp.dot(s, p_ref[...], preferred_element_type=jnp.float32)
    out_ref[...] = (1.0 - _ALPHA) * normf_ref[...] * m + _ALPHA * est0_ref[...]


_step = pl.pallas_call(
    _step_body,
    grid=(_NP // _RB,),
    in_specs=[
        pl.BlockSpec((_NC, _RB, _C), lambda i: (0, i, 0)),
        pl.BlockSpec((_RB, _C), lambda i: (i, 0)),
        pl.BlockSpec((_RB, _C), lambda i: (i, 0)),
        pl.BlockSpec((_C, _C), lambda i: (0, 0)),
    ],
    out_specs=pl.BlockSpec((_RB, _C), lambda i: (i, 0)),
    out_shape=jax.ShapeDtypeStruct((_NP, _C), jnp.float32),
)


def kernel(edge_index, estimates, W):
    src = edge_index[0]
    dst = edge_index[1]
    dst3 = dst.reshape(_NW, _NCHUNK, _K)
    P = _softmax(W)
    degp = _deg(dst3)
    normf = _norm(degp)
    est0 = jnp.pad(estimates, ((0, _NP - _N), (0, 0)))
    est = est0
    for _ in range(_NUM_ITERS):
        sp = _agg(est, src, dst)
        est = _step(sp, normf, est0, P)
    return est[:_N]
